# baseline (speedup 1.0000x reference)
import jax
import jax.numpy as jnp
from jax.experimental import pallas as pl
from jax.experimental.pallas import tpu as pltpu

S = 64.0
M = 0.4

_BR = 3328


def _cosface_block(lab_ref, x_ref, o_ref):
    i = pl.program_id(0)
    lab = lab_ref[...]
    row = jax.lax.broadcasted_iota(jnp.int32, x_ref.shape, 0) + i * _BR
    hit = row == lab
    x = x_ref[...]
    o_ref[...] = x * S - (M * S) * hit.astype(jnp.float32)


def kernel(logits, labels):
    B, C = logits.shape
    lt = logits.T
    lab2 = labels.reshape(1, B)
    out_t = pl.pallas_call(
        _cosface_block,
        grid=(pl.cdiv(C, _BR),),
        in_specs=[
            pl.BlockSpec((1, B), lambda i: (0, 0)),
            pl.BlockSpec((_BR, B), lambda i: (i, 0)),
        ],
        out_specs=pl.BlockSpec((_BR, B), lambda i: (i, 0)),
        out_shape=jax.ShapeDtypeStruct((C, B), logits.dtype),
        compiler_params=pltpu.CompilerParams(
            dimension_semantics=("arbitrary",),
        ),
    )(lab2, lt)
    return out_t.T


# speedup vs baseline: 1.0051x; 1.0051x over previous
import jax
import jax.numpy as jnp
from jax.experimental import pallas as pl
from jax.experimental.pallas import tpu as pltpu

S = 64.0
M = 0.4

_BR = 3072


def _cosface_block(lab_ref, x_ref, o_ref):
    i = pl.program_id(0)
    lab = lab_ref[...]
    row = jax.lax.broadcasted_iota(jnp.int32, x_ref.shape, 0) + i * _BR
    hit = row == lab
    x = x_ref[...]
    o_ref[...] = x * S - (M * S) * hit.astype(jnp.float32)


def kernel(logits, labels):
    B, C = logits.shape
    lt = logits.T
    lab2 = labels.reshape(1, B)
    out_t = pl.pallas_call(
        _cosface_block,
        grid=(pl.cdiv(C, _BR),),
        in_specs=[
            pl.BlockSpec((1, B), lambda i: (0, 0)),
            pl.BlockSpec((_BR, B), lambda i: (i, 0)),
        ],
        out_specs=pl.BlockSpec((_BR, B), lambda i: (i, 0)),
        out_shape=jax.ShapeDtypeStruct((C, B), logits.dtype),
        compiler_params=pltpu.CompilerParams(
            dimension_semantics=("arbitrary",),
        ),
    )(lab2, lt)
    return out_t.T
